# baseline (speedup 1.0000x reference)
import functools

import jax
import jax.numpy as jnp
from jax import lax
from jax.experimental import pallas as pl
from jax.experimental.pallas import tpu as pltpu

_BN_EPS = 1e-5
_IMGS = 8


def _conv_stats_kernel(xph_ref, w_ref, y_ref, stats_ref, scr_ref, *, oh, ow):
    ohw = oh * ow
    cin = xph_ref.shape[-1]
    for g in range(xph_ref.shape[0]):
        b = g % 2
        for kj in range(3):
            skj = xph_ref[g, :, kj:kj + ow, :].reshape((oh + 2) * ow, cin)
            for ki in range(3):
                k = ki * 3 + kj
                scr_ref[b, :, k * cin:(k + 1) * cin] = \
                    skj[ki * ow:ki * ow + ohw, :].astype(jnp.bfloat16)
        acc = jnp.dot(scr_ref[b], w_ref[...],
                      preferred_element_type=jnp.float32)
        stats_ref[g, 0:1, :] = jnp.sum(acc, axis=0, keepdims=True)
        stats_ref[g, 1:2, :] = jnp.sum(acc * acc, axis=0, keepdims=True)
        y_ref[g] = acc.astype(jnp.bfloat16)


def _bn_relu_kernel(y_ref, scale_ref, shift_ref, o_ref):
    y = y_ref[...].astype(jnp.float32)
    o_ref[...] = jnp.maximum(y * scale_ref[...] + shift_ref[...], 0.0)


@jax.jit
def _forward(x_nchw, conv_weight, gamma, beta):
    N, Cin, H, W = x_nchw.shape
    Cout = conv_weight.shape[0]
    OH, OW = H, W
    OHW = OH * OW
    G = _IMGS if N % _IMGS == 0 else 1

    x_nhwc = jnp.transpose(x_nchw, (0, 2, 3, 1))
    xph = jnp.pad(x_nhwc, ((0, 0), (1, 1), (1, 1), (0, 0)))

    w = jnp.transpose(conv_weight, (2, 3, 1, 0)).reshape(9 * Cin, Cout)
    w = w.astype(jnp.bfloat16)

    kernel1 = functools.partial(_conv_stats_kernel, oh=OH, ow=OW)
    flops = 2 * N * OHW * (9 * Cin) * Cout
    bytes_acc = 4 * (xph.size + w.size) + 2 * N * OHW * Cout + 4 * N * 2 * Cout
    y, stats = pl.pallas_call(
        kernel1,
        out_shape=(
            jax.ShapeDtypeStruct((N, OHW, Cout), jnp.bfloat16),
            jax.ShapeDtypeStruct((N, 2, Cout), jnp.float32),
        ),
        grid=(N // G,),
        in_specs=[
            pl.BlockSpec((G, OH + 2, OW + 2, Cin), lambda n: (n, 0, 0, 0)),
            pl.BlockSpec((9 * Cin, Cout), lambda n: (0, 0)),
        ],
        out_specs=(
            pl.BlockSpec((G, OHW, Cout), lambda n: (n, 0, 0)),
            pl.BlockSpec((G, 2, Cout), lambda n: (n, 0, 0)),
        ),
        scratch_shapes=[pltpu.VMEM((2, OHW, 9 * Cin), jnp.bfloat16)],
        compiler_params=pltpu.CompilerParams(dimension_semantics=("parallel",)),
        cost_estimate=pl.CostEstimate(flops=flops, transcendentals=0,
                                      bytes_accessed=bytes_acc),
    )(xph, w)

    count = float(N * OHW)
    ssum = jnp.sum(stats[:, 0, :], axis=0)
    ssq = jnp.sum(stats[:, 1, :], axis=0)
    mean = ssum / count
    var = jnp.maximum(ssq / count - mean * mean, 0.0)
    scale = gamma * lax.rsqrt(var + _BN_EPS)
    shift = beta - mean * scale

    out_flat = pl.pallas_call(
        _bn_relu_kernel,
        out_shape=jax.ShapeDtypeStruct((N, OHW, Cout), jnp.float32),
        grid=(N // G,),
        in_specs=[
            pl.BlockSpec((G, OHW, Cout), lambda n: (n, 0, 0)),
            pl.BlockSpec((1, 1, Cout), lambda n: (0, 0, 0)),
            pl.BlockSpec((1, 1, Cout), lambda n: (0, 0, 0)),
        ],
        out_specs=pl.BlockSpec((G, OHW, Cout), lambda n: (n, 0, 0)),
        compiler_params=pltpu.CompilerParams(dimension_semantics=("parallel",)),
    )(y, scale.reshape(1, 1, Cout), shift.reshape(1, 1, Cout))

    out = out_flat.reshape(N, OH, OW, Cout)
    return jnp.transpose(out, (0, 3, 1, 2))


def kernel(x_nchw, conv_weight, gamma, beta):
    return _forward(x_nchw, conv_weight, gamma, beta)


# speedup vs baseline: 1.3831x; 1.3831x over previous
import functools

import jax
import jax.numpy as jnp
from jax import lax
from jax.experimental import pallas as pl
from jax.experimental.pallas import tpu as pltpu

_BN_EPS = 1e-5
_IMGS = 8


def _conv_stats_kernel(x_ref, w_ref, y_ref, stats_ref, scr_ref, *, oh, ow):
    ohw = oh * ow
    cin = x_ref.shape[-1]
    zrow = jnp.zeros((ow, cin), jnp.float32)
    zcol = jnp.zeros((oh, 1, cin), jnp.float32)
    for g in range(x_ref.shape[0]):
        b = g % 2
        for kj in range(3):
            if kj == 0:
                mid = jnp.concatenate([zcol, x_ref[g, :, :ow - 1, :]], axis=1)
            elif kj == 1:
                mid = x_ref[g]
            else:
                mid = jnp.concatenate([x_ref[g, :, 1:, :], zcol], axis=1)
            skj = jnp.concatenate([zrow, mid.reshape(ohw, cin), zrow], axis=0)
            for ki in range(3):
                k = ki * 3 + kj
                scr_ref[b, :, k * cin:(k + 1) * cin] = \
                    skj[ki * ow:ki * ow + ohw, :].astype(jnp.bfloat16)
        acc = jnp.dot(scr_ref[b], w_ref[...],
                      preferred_element_type=jnp.float32)
        stats_ref[g, 0:1, :] = jnp.sum(acc, axis=0, keepdims=True)
        stats_ref[g, 1:2, :] = jnp.sum(acc * acc, axis=0, keepdims=True)
        y_ref[g] = acc.astype(jnp.bfloat16)


def _bn_relu_kernel(y_ref, scale_ref, shift_ref, o_ref):
    y = y_ref[...].astype(jnp.float32)
    o_ref[...] = jnp.maximum(y * scale_ref[...] + shift_ref[...], 0.0)


@jax.jit
def _forward(x_nchw, conv_weight, gamma, beta):
    N, Cin, H, W = x_nchw.shape
    Cout = conv_weight.shape[0]
    OH, OW = H, W
    OHW = OH * OW
    G = _IMGS if N % _IMGS == 0 else 1

    x_nhwc = jnp.transpose(x_nchw, (0, 2, 3, 1))

    w = jnp.transpose(conv_weight, (2, 3, 1, 0)).reshape(9 * Cin, Cout)
    w = w.astype(jnp.bfloat16)

    kernel1 = functools.partial(_conv_stats_kernel, oh=OH, ow=OW)
    flops = 2 * N * OHW * (9 * Cin) * Cout
    bytes_acc = 4 * x_nhwc.size + 2 * (w.size + N * OHW * Cout) + 4 * N * 2 * Cout
    y, stats = pl.pallas_call(
        kernel1,
        out_shape=(
            jax.ShapeDtypeStruct((N, OHW, Cout), jnp.bfloat16),
            jax.ShapeDtypeStruct((N, 2, Cout), jnp.float32),
        ),
        grid=(N // G,),
        in_specs=[
            pl.BlockSpec((G, OH, OW, Cin), lambda n: (n, 0, 0, 0)),
            pl.BlockSpec((9 * Cin, Cout), lambda n: (0, 0)),
        ],
        out_specs=(
            pl.BlockSpec((G, OHW, Cout), lambda n: (n, 0, 0)),
            pl.BlockSpec((G, 2, Cout), lambda n: (n, 0, 0)),
        ),
        scratch_shapes=[pltpu.VMEM((2, OHW, 9 * Cin), jnp.bfloat16)],
        compiler_params=pltpu.CompilerParams(dimension_semantics=("parallel",)),
        cost_estimate=pl.CostEstimate(flops=flops, transcendentals=0,
                                      bytes_accessed=bytes_acc),
    )(x_nhwc, w)

    count = float(N * OHW)
    ssum = jnp.sum(stats[:, 0, :], axis=0)
    ssq = jnp.sum(stats[:, 1, :], axis=0)
    mean = ssum / count
    var = jnp.maximum(ssq / count - mean * mean, 0.0)
    scale = gamma * lax.rsqrt(var + _BN_EPS)
    shift = beta - mean * scale

    out_flat = pl.pallas_call(
        _bn_relu_kernel,
        out_shape=jax.ShapeDtypeStruct((N, OHW, Cout), jnp.float32),
        grid=(N // G,),
        in_specs=[
            pl.BlockSpec((G, OHW, Cout), lambda n: (n, 0, 0)),
            pl.BlockSpec((1, 1, Cout), lambda n: (0, 0, 0)),
            pl.BlockSpec((1, 1, Cout), lambda n: (0, 0, 0)),
        ],
        out_specs=pl.BlockSpec((G, OHW, Cout), lambda n: (n, 0, 0)),
        compiler_params=pltpu.CompilerParams(dimension_semantics=("parallel",)),
    )(y, scale.reshape(1, 1, Cout), shift.reshape(1, 1, Cout))

    out = out_flat.reshape(N, OH, OW, Cout)
    return jnp.transpose(out, (0, 3, 1, 2))


def kernel(x_nchw, conv_weight, gamma, beta):
    return _forward(x_nchw, conv_weight, gamma, beta)


# speedup vs baseline: 1.3838x; 1.0005x over previous
import functools

import jax
import jax.numpy as jnp
from jax import lax
from jax.experimental import pallas as pl
from jax.experimental.pallas import tpu as pltpu

_BN_EPS = 1e-5
_IMGS = 8


def _conv_stats_kernel(x_ref, w_ref, y_ref, stats_ref, scr_ref, *, oh, ow):
    ohw = oh * ow
    cin = x_ref.shape[-1]
    zrow = jnp.zeros((ow, cin), jnp.float32)
    zcol = jnp.zeros((oh, 1, cin), jnp.float32)
    nb = scr_ref.shape[0]
    for g in range(x_ref.shape[0]):
        b = g % nb
        for kj in range(3):
            if kj == 0:
                mid = jnp.concatenate([zcol, x_ref[g, :, :ow - 1, :]], axis=1)
            elif kj == 1:
                mid = x_ref[g]
            else:
                mid = jnp.concatenate([x_ref[g, :, 1:, :], zcol], axis=1)
            skj = jnp.concatenate([zrow, mid.reshape(ohw, cin), zrow], axis=0)
            for ki in range(3):
                k = ki * 3 + kj
                scr_ref[b, :, k * cin:(k + 1) * cin] = \
                    skj[ki * ow:ki * ow + ohw, :].astype(jnp.bfloat16)
        acc = jnp.dot(scr_ref[b], w_ref[...],
                      preferred_element_type=jnp.float32)
        stats_ref[g, 0:1, :] = jnp.sum(acc, axis=0, keepdims=True)
        stats_ref[g, 1:2, :] = jnp.sum(acc * acc, axis=0, keepdims=True)
        y_ref[g] = acc.astype(jnp.bfloat16)


def _bn_relu_kernel(y_ref, scale_ref, shift_ref, o_ref):
    y = y_ref[...].astype(jnp.float32)
    o_ref[...] = jnp.maximum(y * scale_ref[...] + shift_ref[...], 0.0)


@jax.jit
def _forward(x_nchw, conv_weight, gamma, beta):
    N, Cin, H, W = x_nchw.shape
    Cout = conv_weight.shape[0]
    OH, OW = H, W
    OHW = OH * OW
    G = _IMGS if N % _IMGS == 0 else 1
    G1 = G

    x_nhwc = jnp.transpose(x_nchw, (0, 2, 3, 1))

    w = jnp.transpose(conv_weight, (2, 3, 1, 0)).reshape(9 * Cin, Cout)
    w = w.astype(jnp.bfloat16)

    kernel1 = functools.partial(_conv_stats_kernel, oh=OH, ow=OW)
    flops = 2 * N * OHW * (9 * Cin) * Cout
    bytes_acc = 4 * x_nhwc.size + 2 * (w.size + N * OHW * Cout) + 4 * N * 2 * Cout
    y, stats = pl.pallas_call(
        kernel1,
        out_shape=(
            jax.ShapeDtypeStruct((N, OHW, Cout), jnp.bfloat16),
            jax.ShapeDtypeStruct((N, 2, Cout), jnp.float32),
        ),
        grid=(N // G1,),
        in_specs=[
            pl.BlockSpec((G1, OH, OW, Cin), lambda n: (n, 0, 0, 0)),
            pl.BlockSpec((9 * Cin, Cout), lambda n: (0, 0)),
        ],
        out_specs=(
            pl.BlockSpec((G1, OHW, Cout), lambda n: (n, 0, 0)),
            pl.BlockSpec((G1, 2, Cout), lambda n: (n, 0, 0)),
        ),
        scratch_shapes=[pltpu.VMEM((2, OHW, 9 * Cin), jnp.bfloat16)],
        compiler_params=pltpu.CompilerParams(dimension_semantics=("parallel",)),
        cost_estimate=pl.CostEstimate(flops=flops, transcendentals=0,
                                      bytes_accessed=bytes_acc),
    )(x_nhwc, w)

    count = float(N * OHW)
    ssum = jnp.sum(stats[:, 0, :], axis=0)
    ssq = jnp.sum(stats[:, 1, :], axis=0)
    mean = ssum / count
    var = jnp.maximum(ssq / count - mean * mean, 0.0)
    scale = gamma * lax.rsqrt(var + _BN_EPS)
    shift = beta - mean * scale

    out_flat = pl.pallas_call(
        _bn_relu_kernel,
        out_shape=jax.ShapeDtypeStruct((N, OHW, Cout), jnp.float32),
        grid=(N // G,),
        in_specs=[
            pl.BlockSpec((G, OHW, Cout), lambda n: (n, 0, 0)),
            pl.BlockSpec((1, 1, Cout), lambda n: (0, 0, 0)),
            pl.BlockSpec((1, 1, Cout), lambda n: (0, 0, 0)),
        ],
        out_specs=pl.BlockSpec((G, OHW, Cout), lambda n: (n, 0, 0)),
        compiler_params=pltpu.CompilerParams(dimension_semantics=("parallel",)),
    )(y, scale.reshape(1, 1, Cout), shift.reshape(1, 1, Cout))

    out = out_flat.reshape(N, OH, OW, Cout)
    return jnp.transpose(out, (0, 3, 1, 2))


def kernel(x_nchw, conv_weight, gamma, beta):
    return _forward(x_nchw, conv_weight, gamma, beta)
